# baseline (speedup 1.0000x reference)
import jax
import jax.numpy as jnp
from jax import lax
from jax.experimental import pallas as pl
from jax.experimental.pallas import tpu as pltpu
from jax.experimental.pallas import tpu_sc as plsc

F32 = jnp.float32
BF16 = jnp.bfloat16

B = 2
N = 10000
D = 128
E = 160000
H1 = 64
H2 = 32
ZW = 16

NC = 2
NS = 16
CH = 128
E_PAD = 163840
PER_TILE = E_PAD // NS
CHUNKS = PER_TILE // CH
N_PAD = 10240
ROWS_PER_TILE = N_PAD // NS
RB2 = 512
BE = 4 * RB2
PW = H1 // 2


def _sc_mesh():
    return plsc.VectorSubcoreMesh(core_axis_name="c", subcore_axis_name="s",
                                  num_cores=NC, num_subcores=NS)


def _pack_bf16_words(v):
    u = lax.bitcast_convert_type(v, jnp.uint32)
    ur = u + 0x7FFF + ((u >> 16) & 1)
    h = ur >> 16
    w = h[:, :PW] | (h[:, PW:] << 16)
    return lax.bitcast_convert_type(w, F32)


def _unpack_bf16_words(w):
    u = lax.bitcast_convert_type(w, jnp.uint32)
    lo = lax.bitcast_convert_type(u << 16, F32)
    hi = lax.bitcast_convert_type(u & jnp.uint32(0xFFFF0000), F32)
    return jnp.concatenate([lo, hi], axis=1)


def _tables_body(nf_ref, w1s_ref, w1t_ref, b1_ref, p_ref, q_ref):
    x = nf_ref[0]
    p = jnp.dot(x, w1s_ref[...], preferred_element_type=F32)
    q = jnp.dot(x, w1t_ref[...], preferred_element_type=F32) + b1_ref[...]
    p_ref[0] = _pack_bf16_words(p)
    q_ref[0] = _pack_bf16_words(q)


def _tables(nf, w1s, w1t, b1r):
    nb = 1000
    return pl.pallas_call(
        _tables_body,
        grid=(B, N // nb),
        in_specs=[
            pl.BlockSpec((1, nb, D), lambda b, i: (b, i, 0)),
            pl.BlockSpec((D, H1), lambda b, i: (0, 0)),
            pl.BlockSpec((D, H1), lambda b, i: (0, 0)),
            pl.BlockSpec((1, H1), lambda b, i: (0, 0)),
        ],
        out_specs=[
            pl.BlockSpec((1, nb, PW), lambda b, i: (b, i, 0)),
            pl.BlockSpec((1, nb, PW), lambda b, i: (b, i, 0)),
        ],
        out_shape=[jax.ShapeDtypeStruct((B, N, PW), F32)] * 2,
    )(nf, w1s, w1t, b1r)


KG = 4
NCG = CHUNKS // KG
CG = KG * CH


def _gather_body(pf, qf, srcg, tgtg, gs, gt, idxs, idxt,
                 bap, baq, bbp, bbq, gsa, gsb, wsa, wsb):
    c = lax.axis_index("c")
    s = lax.axis_index("s")
    pltpu.sync_copy(srcg.at[c, s], idxs)
    pltpu.sync_copy(tgtg.at[c, s], idxt)
    base = c * E_PAD + s * PER_TILE

    def start(j, bp, bq, sem):
        pltpu.async_copy(pf.at[idxs.at[pl.ds(j * CG, CG)]], bp, sem)
        pltpu.async_copy(qf.at[idxt.at[pl.ds(j * CG, CG)]], bq, sem)

    def drain_gather(bp, bq, sem):
        pltpu.make_async_copy(pf.at[idxs.at[pl.ds(0, CG)]], bp, sem).wait()
        pltpu.make_async_copy(qf.at[idxt.at[pl.ds(0, CG)]], bq, sem).wait()

    rbase = base // 4
    rcg = CG // 4

    def start_wb(j, bp, bq, sem):
        off = rbase + j * rcg
        for t in range(4):
            rows = pl.ds(rcg * t, rcg)
            cols = pl.ds(PW * t, PW)
            pltpu.async_copy(bp.at[rows], gs.at[pl.ds(off, rcg), cols], sem)
            pltpu.async_copy(bq.at[rows], gt.at[pl.ds(off, rcg), cols], sem)

    def drain_wb(bp, bq, sem):
        for t in range(4):
            rows = pl.ds(rcg * t, rcg)
            cols = pl.ds(PW * t, PW)
            pltpu.make_async_copy(bp.at[rows], gs.at[pl.ds(rbase, rcg), cols],
                                  sem).wait()
            pltpu.make_async_copy(bq.at[rows], gt.at[pl.ds(rbase, rcg), cols],
                                  sem).wait()

    start(0, bap, baq, gsa)

    def body(g, carry):
        j0 = 2 * g

        @pl.when(g > 0)
        def _():
            drain_wb(bbp, bbq, wsb)

        start(j0 + 1, bbp, bbq, gsb)
        drain_gather(bap, baq, gsa)
        start_wb(j0, bap, baq, wsa)

        @pl.when(g < NCG // 2 - 1)
        def _():
            drain_wb(bap, baq, wsa)
            start(j0 + 2, bap, baq, gsa)

        drain_gather(bbp, bbq, gsb)
        start_wb(j0 + 1, bbp, bbq, wsb)
        return carry

    lax.fori_loop(0, NCG // 2, body, 0)
    drain_wb(bap, baq, wsa)
    drain_wb(bbp, bbq, wsb)


def _gather(pf, qf, src_g4, tgt_g4):
    return pl.kernel(
        _gather_body,
        out_type=[jax.ShapeDtypeStruct((B * E_PAD // 4, D), F32)] * 2,
        mesh=_sc_mesh(),
        scratch_types=[
            pltpu.VMEM((PER_TILE,), jnp.int32),
            pltpu.VMEM((PER_TILE,), jnp.int32),
            pltpu.VMEM((CG, PW), F32),
            pltpu.VMEM((CG, PW), F32),
            pltpu.VMEM((CG, PW), F32),
            pltpu.VMEM((CG, PW), F32),
            pltpu.SemaphoreType.DMA,
            pltpu.SemaphoreType.DMA,
            pltpu.SemaphoreType.DMA,
            pltpu.SemaphoreType.DMA,
        ],
        compiler_params=pltpu.CompilerParams(use_tc_tiling_on_sc=False),
    )(pf, qf, src_g4, tgt_g4)


def _edge_body(gs_ref, gt_ref, ew4_ref, w1e_ref, w2_ref, b2_ref,
               w3_ref, b3_ref, w4_ref, z_ref, zn_ref):
    gs = gs_ref[0]
    gt = gt_ref[0]
    ew4 = ew4_ref[0, 0]
    w1e = w1e_ref[...]
    nr = RB2 // 2
    eye = (lax.broadcasted_iota(jnp.int32, (D, D), 0)
           == lax.broadcasted_iota(jnp.int32, (D, D), 1)).astype(F32)
    xs = []
    for p in range(2):
        rows = slice(nr * p, nr * (p + 1))
        for t in range(4):
            k = 4 * p + t
            cols = slice(PW * t, PW * (t + 1))
            g = _unpack_bf16_words(gs[rows, cols]) + _unpack_bf16_words(gt[rows, cols])
            tk = lax.dot_general(eye, ew4[k], (((1,), (1,)), ((), ())),
                                 preferred_element_type=F32)
            vcol = jnp.concatenate([tk[:, 0:1], tk[:, 1:2]], axis=0)
            xs.append(g + vcol * w1e)
    h1 = jax.nn.relu(jnp.concatenate(xs, axis=0))
    h2 = jax.nn.relu(jnp.dot(h1, w2_ref[...], preferred_element_type=F32) + b2_ref[...])
    e3 = jax.nn.sigmoid(jnp.dot(h2, w3_ref[...], preferred_element_type=F32) + b3_ref[...])
    z = jnp.dot(e3, w4_ref[...], preferred_element_type=F32)
    out = jnp.concatenate([z[nr * k:nr * (k + 1)] for k in range(8)], axis=1)
    z_ref[0] = out
    zn_ref[0] = -out


def _edge_mlp(gsr, gtr, ew4, w1e, w2, b2r, w3, b3r, w4a):
    return pl.pallas_call(
        _edge_body,
        grid=(B, E_PAD // BE),
        in_specs=[
            pl.BlockSpec((1, RB2, D), lambda b, i: (b, i, 0)),
            pl.BlockSpec((1, RB2, D), lambda b, i: (b, i, 0)),
            pl.BlockSpec((1, 1, 8, 2, D), lambda b, i: (b, i, 0, 0, 0)),
            pl.BlockSpec((1, H1), lambda b, i: (0, 0)),
            pl.BlockSpec((H1, H2), lambda b, i: (0, 0)),
            pl.BlockSpec((1, H2), lambda b, i: (0, 0)),
            pl.BlockSpec((H2, H1), lambda b, i: (0, 0)),
            pl.BlockSpec((1, H1), lambda b, i: (0, 0)),
            pl.BlockSpec((H1, ZW), lambda b, i: (0, 0)),
        ],
        out_specs=[
            pl.BlockSpec((1, RB2 // 2, 8 * ZW), lambda b, i: (b, i, 0)),
            pl.BlockSpec((1, RB2 // 2, 8 * ZW), lambda b, i: (b, i, 0)),
        ],
        out_shape=[jax.ShapeDtypeStruct((B, E_PAD // 8, 8 * ZW), F32)] * 2,
    )(gsr, gtr, ew4, w1e, w2, b2r, w3, b3r, w4a)


KS = 4
NCS = CHUNKS // KS
CS = KS * CH


def _scatter_body(zf, znf, tgts, srcs, accout, idx1, idx2,
                  zba1, zba2, zbb1, zbb2, zrows, acc_sh, lsa, lsb, ssa, ssb):
    c = lax.axis_index("c")
    s = lax.axis_index("s")
    pltpu.sync_copy(tgts.at[s], idx1)
    pltpu.sync_copy(srcs.at[s], idx2)

    def zero_row(i, carry):
        zrows[i] = jnp.zeros((ZW,), F32)
        return carry

    lax.fori_loop(0, CH, zero_row, 0)
    for k in range(ROWS_PER_TILE // CH):
        pltpu.sync_copy(zrows, acc_sh.at[pl.ds(s * ROWS_PER_TILE + k * CH, CH)])
    plsc.subcore_barrier()

    base = c * E_PAD + s * PER_TILE

    def load(j, b1, b2, sem):
        off = base + j * CS
        pltpu.async_copy(zf.at[pl.ds(off, CS)], b1, sem)
        pltpu.async_copy(znf.at[pl.ds(off, CS)], b2, sem)

    def drain_load(b1, b2, sem):
        pltpu.make_async_copy(zf.at[pl.ds(base, CS)], b1, sem).wait()
        pltpu.make_async_copy(znf.at[pl.ds(base, CS)], b2, sem).wait()

    def scat(j, b1, b2, sem):
        pltpu.async_copy(b1, acc_sh.at[idx1.at[pl.ds(j * CS, CS)]], sem, add=True)
        pltpu.async_copy(b2, acc_sh.at[idx2.at[pl.ds(j * CS, CS)]], sem, add=True)

    def drain_scat(b1, b2, sem):
        pltpu.make_async_copy(b1, acc_sh.at[idx1.at[pl.ds(0, CS)]], sem).wait()
        pltpu.make_async_copy(b2, acc_sh.at[idx2.at[pl.ds(0, CS)]], sem).wait()

    load(0, zba1, zba2, lsa)

    def body(g, carry):
        j0 = 2 * g

        @pl.when(g > 0)
        def _():
            drain_scat(zbb1, zbb2, ssb)

        load(j0 + 1, zbb1, zbb2, lsb)
        drain_load(zba1, zba2, lsa)
        scat(j0, zba1, zba2, ssa)

        @pl.when(g < NCS // 2 - 1)
        def _():
            drain_scat(zba1, zba2, ssa)
            load(j0 + 2, zba1, zba2, lsa)

        drain_load(zbb1, zbb2, lsb)
        scat(j0 + 1, zbb1, zbb2, ssb)
        return carry

    lax.fori_loop(0, NCS // 2, body, 0)
    drain_scat(zba1, zba2, ssa)
    drain_scat(zbb1, zbb2, ssb)
    plsc.subcore_barrier()
    pltpu.sync_copy(
        acc_sh.at[pl.ds(s * ROWS_PER_TILE, ROWS_PER_TILE)],
        accout.at[pl.ds(c * N_PAD + s * ROWS_PER_TILE, ROWS_PER_TILE)],
    )


def _scatter(zf, znf, tgt_s3, src_s3):
    return pl.kernel(
        _scatter_body,
        out_type=jax.ShapeDtypeStruct((B * N_PAD, ZW), F32),
        mesh=_sc_mesh(),
        scratch_types=[
            pltpu.VMEM((PER_TILE,), jnp.int32),
            pltpu.VMEM((PER_TILE,), jnp.int32),
            pltpu.VMEM((CS, ZW), F32),
            pltpu.VMEM((CS, ZW), F32),
            pltpu.VMEM((CS, ZW), F32),
            pltpu.VMEM((CS, ZW), F32),
            pltpu.VMEM((CH, ZW), F32),
            pltpu.VMEM_SHARED((N_PAD, ZW), F32),
            pltpu.SemaphoreType.DMA,
            pltpu.SemaphoreType.DMA,
            pltpu.SemaphoreType.DMA,
            pltpu.SemaphoreType.DMA,
        ],
        compiler_params=pltpu.CompilerParams(use_tc_tiling_on_sc=False),
    )(zf, znf, tgt_s3, src_s3)


def _node_body(acc_ref, lp_ref, w4l_ref, b4_ref, w5_ref, b5_ref, y_ref):
    a = acc_ref[0][:, :12]
    h4 = jax.nn.relu(a + lp_ref[0] * w4l_ref[...] + b4_ref[...])
    y_ref[0] = jax.nn.sigmoid(jnp.dot(h4, w5_ref[...], preferred_element_type=F32)
                              + b5_ref[...])


def _node_mlp(acc, lp, w4l, b4r, w5, b5r):
    nb = 1000
    return pl.pallas_call(
        _node_body,
        grid=(B, N // nb),
        in_specs=[
            pl.BlockSpec((1, nb, ZW), lambda b, i: (b, i, 0)),
            pl.BlockSpec((1, nb, 1), lambda b, i: (b, i, 0)),
            pl.BlockSpec((1, 12), lambda b, i: (0, 0)),
            pl.BlockSpec((1, 12), lambda b, i: (0, 0)),
            pl.BlockSpec((12, 1), lambda b, i: (0, 0)),
            pl.BlockSpec((1, 1), lambda b, i: (0, 0)),
        ],
        out_specs=pl.BlockSpec((1, nb, 1), lambda b, i: (b, i, 0)),
        out_shape=jax.ShapeDtypeStruct((B, N, 1), F32),
    )(acc, lp, w4l, b4r, w5, b5r)


def kernel(node_features, edge_weight, label_prev, edge_index,
           W1, b1, W2, b2, W3, b3, W4, b4, W5, b5):
    src = edge_index[0]
    tgt = edge_index[1]
    pad = E_PAD - E

    src_p = jnp.concatenate([src, jnp.zeros((pad,), jnp.int32)])
    tgt_p = jnp.concatenate([tgt, jnp.zeros((pad,), jnp.int32)])
    def chunk_perm(v):
        return v.reshape(-1, CG // 4, 4).transpose(0, 2, 1).reshape(-1)

    src_p = chunk_perm(src_p)
    tgt_p = chunk_perm(tgt_p)
    src_g4 = jnp.concatenate([src_p, src_p + N]).reshape(NC, NS, PER_TILE)
    tgt_g4 = jnp.concatenate([tgt_p, tgt_p + N]).reshape(NC, NS, PER_TILE)

    def z_perm(v):
        return (v.reshape(-1, 2, BE // 8, 4).transpose(0, 2, 1, 3)
                .reshape(NS, PER_TILE))

    dump = jnp.full((pad,), N, jnp.int32)
    src_s3 = z_perm(jnp.concatenate([src, dump]))
    tgt_s3 = z_perm(jnp.concatenate([tgt, dump]))

    ew_p = jnp.pad(edge_weight, ((0, 0), (0, pad)))
    nr = RB2 // 2
    ew8 = (ew_p.reshape(B, E_PAD // BE, 2, 2, D, 4)
           .transpose(0, 1, 2, 5, 3, 4)
           .reshape(B, E_PAD // BE, 8, 2, D))

    w1s = W1[:D]
    w1t = W1[D:2 * D]
    w1e = W1[2 * D].reshape(1, H1)
    w4a = jnp.pad(W4[:H1], ((0, 0), (0, ZW - 12)))
    w4l = W4[H1].reshape(1, 12)

    p, q = _tables(node_features, w1s, w1t, b1.reshape(1, H1))
    gs, gt = _gather(p.reshape(B * N, PW), q.reshape(B * N, PW), src_g4, tgt_g4)
    z, zn = _edge_mlp(gs.reshape(B, E_PAD // 4, D), gt.reshape(B, E_PAD // 4, D),
                      ew8, w1e, W2, b2.reshape(1, H2), W3,
                      b3.reshape(1, H1), w4a)
    acc = _scatter(z.reshape(B * E_PAD, ZW), zn.reshape(B * E_PAD, ZW),
                   tgt_s3, src_s3)
    y = _node_mlp(acc.reshape(B, N_PAD, ZW), label_prev, w4l,
                  b4.reshape(1, 12), W5, b5.reshape(1, 1))
    return y


# speedup vs baseline: 1.3441x; 1.3441x over previous
import jax
import jax.numpy as jnp
from jax import lax
from jax.experimental import pallas as pl
from jax.experimental.pallas import tpu as pltpu
from jax.experimental.pallas import tpu_sc as plsc

F32 = jnp.float32
BF16 = jnp.bfloat16

B = 2
N = 10000
D = 128
E = 160000
H1 = 64
H2 = 32
ZW = 16

NC = 2
NS = 16
CH = 128
E_PAD = 163840
PER_TILE = E_PAD // NS
CHUNKS = PER_TILE // CH
N_PAD = 10240
ROWS_PER_TILE = N_PAD // NS
RB2 = 512
BE = 4 * RB2
PW = H1 // 2


def _sc_mesh():
    return plsc.VectorSubcoreMesh(core_axis_name="c", subcore_axis_name="s",
                                  num_cores=NC, num_subcores=NS)


def _pack_bf16_words(v):
    u = lax.bitcast_convert_type(v, jnp.uint32)
    ur = u + 0x7FFF + ((u >> 16) & 1)
    h = ur >> 16
    w = h[:, :PW] | (h[:, PW:] << 16)
    return lax.bitcast_convert_type(w, F32)


def _unpack_bf16_words(w):
    u = lax.bitcast_convert_type(w, jnp.uint32)
    lo = lax.bitcast_convert_type(u << 16, F32)
    hi = lax.bitcast_convert_type(u & jnp.uint32(0xFFFF0000), F32)
    return jnp.concatenate([lo, hi], axis=1)


def _tables_body(nf_ref, w1s_ref, w1t_ref, b1_ref, p_ref, q_ref):
    x = nf_ref[0]
    p = jnp.dot(x, w1s_ref[...], preferred_element_type=F32)
    q = jnp.dot(x, w1t_ref[...], preferred_element_type=F32) + b1_ref[...]
    p_ref[0] = _pack_bf16_words(p)
    q_ref[0] = _pack_bf16_words(q)


def _tables(nf, w1s, w1t, b1r):
    nb = 1000
    return pl.pallas_call(
        _tables_body,
        grid=(B, N // nb),
        in_specs=[
            pl.BlockSpec((1, nb, D), lambda b, i: (b, i, 0)),
            pl.BlockSpec((D, H1), lambda b, i: (0, 0)),
            pl.BlockSpec((D, H1), lambda b, i: (0, 0)),
            pl.BlockSpec((1, H1), lambda b, i: (0, 0)),
        ],
        out_specs=[
            pl.BlockSpec((1, nb, PW), lambda b, i: (b, i, 0)),
            pl.BlockSpec((1, nb, PW), lambda b, i: (b, i, 0)),
        ],
        out_shape=[jax.ShapeDtypeStruct((B, N, PW), F32)] * 2,
    )(nf, w1s, w1t, b1r)


KG = 4
NCG = CHUNKS // KG
CG = KG * CH


def _gather_body(pf, qf, srcg, tgtg, gs, gt, idxs, idxt,
                 bap, baq, bbp, bbq, gsa, gsb, wsa, wsb):
    c = lax.axis_index("c")
    s = lax.axis_index("s")
    pltpu.sync_copy(srcg.at[c, s], idxs)
    pltpu.sync_copy(tgtg.at[c, s], idxt)
    base = c * E_PAD + s * PER_TILE

    def start(j, bp, bq, sem):
        pltpu.async_copy(pf.at[idxs.at[pl.ds(j * CG, CG)]], bp, sem)
        pltpu.async_copy(qf.at[idxt.at[pl.ds(j * CG, CG)]], bq, sem)

    def drain_gather(bp, bq, sem):
        pltpu.make_async_copy(pf.at[idxs.at[pl.ds(0, CG)]], bp, sem).wait()
        pltpu.make_async_copy(qf.at[idxt.at[pl.ds(0, CG)]], bq, sem).wait()

    rbase = base // 4
    rcg = CG // 4

    def start_wb(j, bp, bq, sem):
        off = rbase + j * rcg
        for t in range(4):
            rows = pl.ds(rcg * t, rcg)
            cols = pl.ds(PW * t, PW)
            pltpu.async_copy(bp.at[rows], gs.at[pl.ds(off, rcg), cols], sem)
            pltpu.async_copy(bq.at[rows], gt.at[pl.ds(off, rcg), cols], sem)

    def drain_wb(bp, bq, sem):
        for t in range(4):
            rows = pl.ds(rcg * t, rcg)
            cols = pl.ds(PW * t, PW)
            pltpu.make_async_copy(bp.at[rows], gs.at[pl.ds(rbase, rcg), cols],
                                  sem).wait()
            pltpu.make_async_copy(bq.at[rows], gt.at[pl.ds(rbase, rcg), cols],
                                  sem).wait()

    start(0, bap, baq, gsa)

    def body(g, carry):
        j0 = 2 * g

        @pl.when(g > 0)
        def _():
            drain_wb(bbp, bbq, wsb)

        start(j0 + 1, bbp, bbq, gsb)
        drain_gather(bap, baq, gsa)
        start_wb(j0, bap, baq, wsa)

        @pl.when(g < NCG // 2 - 1)
        def _():
            drain_wb(bap, baq, wsa)
            start(j0 + 2, bap, baq, gsa)

        drain_gather(bbp, bbq, gsb)
        start_wb(j0 + 1, bbp, bbq, wsb)
        return carry

    lax.fori_loop(0, NCG // 2, body, 0)
    drain_wb(bap, baq, wsa)
    drain_wb(bbp, bbq, wsb)


def _gather(pf, qf, src_g4, tgt_g4):
    return pl.kernel(
        _gather_body,
        out_type=[jax.ShapeDtypeStruct((B * E_PAD // 4, D), F32)] * 2,
        mesh=_sc_mesh(),
        scratch_types=[
            pltpu.VMEM((PER_TILE,), jnp.int32),
            pltpu.VMEM((PER_TILE,), jnp.int32),
            pltpu.VMEM((CG, PW), F32),
            pltpu.VMEM((CG, PW), F32),
            pltpu.VMEM((CG, PW), F32),
            pltpu.VMEM((CG, PW), F32),
            pltpu.SemaphoreType.DMA,
            pltpu.SemaphoreType.DMA,
            pltpu.SemaphoreType.DMA,
            pltpu.SemaphoreType.DMA,
        ],
        compiler_params=pltpu.CompilerParams(use_tc_tiling_on_sc=False),
    )(pf, qf, src_g4, tgt_g4)


def _edge_body(gs_ref, gt_ref, ew4_ref, w1e_ref, w2_ref, b2_ref,
               w3_ref, b3_ref, w4_ref, z_ref, zn_ref):
    gs = gs_ref[0]
    gt = gt_ref[0]
    ew4 = ew4_ref[0, 0]
    w1e = w1e_ref[...]
    nr = RB2 // 2
    eye = (lax.broadcasted_iota(jnp.int32, (D, D), 0)
           == lax.broadcasted_iota(jnp.int32, (D, D), 1)).astype(F32)
    xs = []
    for p in range(2):
        rows = slice(nr * p, nr * (p + 1))
        for t in range(4):
            k = 4 * p + t
            cols = slice(PW * t, PW * (t + 1))
            g = _unpack_bf16_words(gs[rows, cols]) + _unpack_bf16_words(gt[rows, cols])
            tk = lax.dot_general(eye, ew4[k], (((1,), (1,)), ((), ())),
                                 preferred_element_type=F32)
            vcol = jnp.concatenate([tk[:, 0:1], tk[:, 1:2]], axis=0)
            xs.append(g + vcol * w1e)
    h1 = jax.nn.relu(jnp.concatenate(xs, axis=0))
    h2 = jax.nn.relu(jnp.dot(h1, w2_ref[...], preferred_element_type=F32) + b2_ref[...])
    e3 = jax.nn.sigmoid(jnp.dot(h2, w3_ref[...], preferred_element_type=F32) + b3_ref[...])
    z = jnp.dot(e3, w4_ref[...], preferred_element_type=F32)
    out = jnp.concatenate([z[nr * k:nr * (k + 1)] for k in range(8)], axis=1)
    z_ref[0] = out
    zn_ref[0] = -out


def _edge_mlp(gsr, gtr, ew4, w1e, w2, b2r, w3, b3r, w4a):
    return pl.pallas_call(
        _edge_body,
        grid=(B, E_PAD // BE),
        in_specs=[
            pl.BlockSpec((1, RB2, D), lambda b, i: (b, i, 0)),
            pl.BlockSpec((1, RB2, D), lambda b, i: (b, i, 0)),
            pl.BlockSpec((1, 1, 8, 2, D), lambda b, i: (b, i, 0, 0, 0)),
            pl.BlockSpec((1, H1), lambda b, i: (0, 0)),
            pl.BlockSpec((H1, H2), lambda b, i: (0, 0)),
            pl.BlockSpec((1, H2), lambda b, i: (0, 0)),
            pl.BlockSpec((H2, H1), lambda b, i: (0, 0)),
            pl.BlockSpec((1, H1), lambda b, i: (0, 0)),
            pl.BlockSpec((H1, ZW), lambda b, i: (0, 0)),
        ],
        out_specs=[
            pl.BlockSpec((1, RB2 // 2, 8 * ZW), lambda b, i: (b, i, 0)),
            pl.BlockSpec((1, RB2 // 2, 8 * ZW), lambda b, i: (b, i, 0)),
        ],
        out_shape=[jax.ShapeDtypeStruct((B, E_PAD // 8, 8 * ZW), F32)] * 2,
    )(gsr, gtr, ew4, w1e, w2, b2r, w3, b3r, w4a)


KS = 4
NCS = CHUNKS // KS
CS = KS * CH


def _scatter_body(zf, znf, tgts, srcs, accout, idx1, idx2,
                  zba1, zba2, zbb1, zbb2, zrows, acc_sh, lsa, lsb, ssa, ssb):
    c = lax.axis_index("c")
    s = lax.axis_index("s")
    pltpu.sync_copy(tgts.at[s], idx1)
    pltpu.sync_copy(srcs.at[s], idx2)

    def zero_row(i, carry):
        zrows[i] = jnp.zeros((ZW,), F32)
        return carry

    lax.fori_loop(0, CH, zero_row, 0)
    for k in range(ROWS_PER_TILE // CH):
        pltpu.sync_copy(zrows, acc_sh.at[pl.ds(s * ROWS_PER_TILE + k * CH, CH)])
    plsc.subcore_barrier()

    base = c * E_PAD + s * PER_TILE

    def load(j, b1, b2, sem):
        off = base + j * CS
        pltpu.async_copy(zf.at[pl.ds(off, CS)], b1, sem)
        pltpu.async_copy(znf.at[pl.ds(off, CS)], b2, sem)

    def drain_load(b1, b2, sem):
        pltpu.make_async_copy(zf.at[pl.ds(base, CS)], b1, sem).wait()
        pltpu.make_async_copy(znf.at[pl.ds(base, CS)], b2, sem).wait()

    def scat(j, b1, b2, sem):
        pltpu.async_copy(b1, acc_sh.at[idx1.at[pl.ds(j * CS, CS)]], sem, add=True)
        pltpu.async_copy(b2, acc_sh.at[idx2.at[pl.ds(j * CS, CS)]], sem, add=True)

    def drain_scat(b1, b2, sem):
        pltpu.make_async_copy(b1, acc_sh.at[idx1.at[pl.ds(0, CS)]], sem).wait()
        pltpu.make_async_copy(b2, acc_sh.at[idx2.at[pl.ds(0, CS)]], sem).wait()

    load(0, zba1, zba2, lsa)

    def body(g, carry):
        j0 = 2 * g

        @pl.when(g > 0)
        def _():
            drain_scat(zbb1, zbb2, ssb)

        load(j0 + 1, zbb1, zbb2, lsb)
        drain_load(zba1, zba2, lsa)
        scat(j0, zba1, zba2, ssa)

        @pl.when(g < NCS // 2 - 1)
        def _():
            drain_scat(zba1, zba2, ssa)
            load(j0 + 2, zba1, zba2, lsa)

        drain_load(zbb1, zbb2, lsb)
        scat(j0 + 1, zbb1, zbb2, ssb)
        return carry

    lax.fori_loop(0, NCS // 2, body, 0)
    drain_scat(zba1, zba2, ssa)
    drain_scat(zbb1, zbb2, ssb)
    plsc.subcore_barrier()
    pltpu.sync_copy(
        acc_sh.at[pl.ds(s * ROWS_PER_TILE, ROWS_PER_TILE)],
        accout.at[pl.ds(c * N_PAD + s * ROWS_PER_TILE, ROWS_PER_TILE)],
    )


def _scatter(zf, znf, tgt_s3, src_s3):
    return pl.kernel(
        _scatter_body,
        out_type=jax.ShapeDtypeStruct((B * N_PAD, ZW), F32),
        mesh=_sc_mesh(),
        scratch_types=[
            pltpu.VMEM((PER_TILE,), jnp.int32),
            pltpu.VMEM((PER_TILE,), jnp.int32),
            pltpu.VMEM((CS, ZW), F32),
            pltpu.VMEM((CS, ZW), F32),
            pltpu.VMEM((CS, ZW), F32),
            pltpu.VMEM((CS, ZW), F32),
            pltpu.VMEM((CH, ZW), F32),
            pltpu.VMEM_SHARED((N_PAD, ZW), F32),
            pltpu.SemaphoreType.DMA,
            pltpu.SemaphoreType.DMA,
            pltpu.SemaphoreType.DMA,
            pltpu.SemaphoreType.DMA,
        ],
        compiler_params=pltpu.CompilerParams(use_tc_tiling_on_sc=False),
    )(zf, znf, tgt_s3, src_s3)


def _node_body(acc_ref, lp_ref, w4l_ref, b4_ref, w5_ref, b5_ref, y_ref):
    a = acc_ref[0][:, :12]
    h4 = jax.nn.relu(a + lp_ref[0] * w4l_ref[...] + b4_ref[...])
    y_ref[0] = jax.nn.sigmoid(jnp.dot(h4, w5_ref[...], preferred_element_type=F32)
                              + b5_ref[...])


def _node_mlp(acc, lp, w4l, b4r, w5, b5r):
    nb = 1000
    return pl.pallas_call(
        _node_body,
        grid=(B, N // nb),
        in_specs=[
            pl.BlockSpec((1, nb, ZW), lambda b, i: (b, i, 0)),
            pl.BlockSpec((1, nb, 1), lambda b, i: (b, i, 0)),
            pl.BlockSpec((1, 12), lambda b, i: (0, 0)),
            pl.BlockSpec((1, 12), lambda b, i: (0, 0)),
            pl.BlockSpec((12, 1), lambda b, i: (0, 0)),
            pl.BlockSpec((1, 1), lambda b, i: (0, 0)),
        ],
        out_specs=pl.BlockSpec((1, nb, 1), lambda b, i: (b, i, 0)),
        out_shape=jax.ShapeDtypeStruct((B, N, 1), F32),
    )(acc, lp, w4l, b4r, w5, b5r)


def kernel(node_features, edge_weight, label_prev, edge_index,
           W1, b1, W2, b2, W3, b3, W4, b4, W5, b5):
    src = edge_index[0]
    tgt = edge_index[1]
    pad = E_PAD - E

    src_p = jnp.concatenate([src, jnp.zeros((pad,), jnp.int32)])
    tgt_p = jnp.concatenate([tgt, jnp.zeros((pad,), jnp.int32)])
    gi = jnp.arange(E_PAD, dtype=jnp.int32)
    ii = gi % CG
    gp = (gi - ii) + 4 * (ii % (CG // 4)) + ii // (CG // 4)
    src_p = jnp.take(src_p, gp)
    tgt_p = jnp.take(tgt_p, gp)
    src_g4 = jnp.concatenate([src_p, src_p + N]).reshape(NC, NS, PER_TILE)
    tgt_g4 = jnp.concatenate([tgt_p, tgt_p + N]).reshape(NC, NS, PER_TILE)

    r = jnp.arange(E_PAD, dtype=jnp.int32)
    k = r % 8
    eo = (BE * (r // BE) + 2 * RB2 * (k // 4)
          + 4 * ((r % BE) // 8) + k % 4)
    dump = jnp.full((pad,), N, jnp.int32)
    src_s3 = jnp.take(jnp.concatenate([src, dump]), eo).reshape(NS, PER_TILE)
    tgt_s3 = jnp.take(jnp.concatenate([tgt, dump]), eo).reshape(NS, PER_TILE)

    ew_p = jnp.pad(edge_weight, ((0, 0), (0, pad)))
    nr = RB2 // 2
    ew8 = (ew_p.reshape(B, E_PAD // BE, 2, 2, D, 4)
           .transpose(0, 1, 2, 5, 3, 4)
           .reshape(B, E_PAD // BE, 8, 2, D))

    w1s = W1[:D]
    w1t = W1[D:2 * D]
    w1e = W1[2 * D].reshape(1, H1)
    w4a = jnp.pad(W4[:H1], ((0, 0), (0, ZW - 12)))
    w4l = W4[H1].reshape(1, 12)

    p, q = _tables(node_features, w1s, w1t, b1.reshape(1, H1))
    gs, gt = _gather(p.reshape(B * N, PW), q.reshape(B * N, PW), src_g4, tgt_g4)
    z, zn = _edge_mlp(gs.reshape(B, E_PAD // 4, D), gt.reshape(B, E_PAD // 4, D),
                      ew8, w1e, W2, b2.reshape(1, H2), W3,
                      b3.reshape(1, H1), w4a)
    acc = _scatter(z.reshape(B * E_PAD, ZW), zn.reshape(B * E_PAD, ZW),
                   tgt_s3, src_s3)
    y = _node_mlp(acc.reshape(B, N_PAD, ZW), label_prev, w4l,
                  b4.reshape(1, 12), W5, b5.reshape(1, 1))
    return y
